# Initial kernel scaffold; baseline (speedup 1.0000x reference)
import jax
import jax.numpy as jnp
from jax.experimental import pallas as pl


def kernel(x, edge_index, W1, b1, W2, b2, g1, be1, g2, be2, g3, be3, fc1W, fc1b, fc2W, fc2b):
    raise NotImplementedError("write your pallas kernel here")


# speedup vs baseline: 8.3714x; 8.3714x over previous
import functools
import jax
import jax.numpy as jnp
from jax import lax
from jax.experimental import pallas as pl
from jax.experimental.pallas import tpu as pltpu
from jax.experimental.pallas import tpu_sc as plsc

N = 10000
NP = 10240
E = 160000
EPS = 1e-5
NINV = 1.0 / N

EP = 163840
NWK = 32
EW = EP // NWK
CW = 256
NWIN = EW // CW
RPT = NP // 16


def _t1_body(deg0, deg1, xpad, g1, be1, dinv_o, xp_o, s1_o, t1_o, asum, asq):
    i = pl.program_id(0)
    B = xpad.shape[0]
    deg = deg0[...] + deg1[...] + 1.0
    dinv = lax.rsqrt(deg)
    dinv_o[...] = dinv
    x = xpad[...]
    col = lax.broadcasted_iota(jnp.int32, (B, 128), 1)
    xp_o[...] = jnp.where(col == 127, dinv, dinv * x)

    @pl.when(i == 0)
    def _():
        asum[...] = jnp.zeros_like(asum)
        asq[...] = jnp.zeros_like(asq)

    asum[...] += jnp.sum(x, axis=0, keepdims=True)
    asq[...] += jnp.sum(x * x, axis=0, keepdims=True)

    @pl.when(i == pl.num_programs(0) - 1)
    def _():
        mu = asum[...] * NINV
        var = asq[...] * NINV - mu * mu
        s1 = g1[...] * lax.rsqrt(var + EPS)
        s1_o[...] = s1
        t1_o[...] = be1[...] - mu * s1


def _t1_call(deg0, deg1, xpad, g1, be1):
    B = 512
    grid = NP // B
    return pl.pallas_call(
        _t1_body,
        grid=(grid,),
        in_specs=[
            pl.BlockSpec((B, 1), lambda i: (i, 0)),
            pl.BlockSpec((B, 1), lambda i: (i, 0)),
            pl.BlockSpec((B, 128), lambda i: (i, 0)),
            pl.BlockSpec((1, 128), lambda i: (0, 0)),
            pl.BlockSpec((1, 128), lambda i: (0, 0)),
        ],
        out_specs=[
            pl.BlockSpec((B, 1), lambda i: (i, 0)),
            pl.BlockSpec((B, 128), lambda i: (i, 0)),
            pl.BlockSpec((1, 128), lambda i: (0, 0)),
            pl.BlockSpec((1, 128), lambda i: (0, 0)),
        ],
        out_shape=[
            jax.ShapeDtypeStruct((NP, 1), jnp.float32),
            jax.ShapeDtypeStruct((NP, 128), jnp.float32),
            jax.ShapeDtypeStruct((1, 128), jnp.float32),
            jax.ShapeDtypeStruct((1, 128), jnp.float32),
        ],
        scratch_shapes=[
            pltpu.VMEM((1, 128), jnp.float32),
            pltpu.VMEM((1, 128), jnp.float32),
        ],
    )(deg0, deg1, xpad, g1, be1)


def _t3_body(g1a, g1b, dinv, w1, s1, t1, b1, g2, be2,
             z1_o, rs_o, s2_o, t2_o, asum, asq):
    i = pl.program_id(0)
    B = g1a.shape[0]
    dv = dinv[...]
    g = g1a[...] + g1b[...]
    rs = g[:, 127:128]
    rs_o[...] = rs
    lhs = g * dv * s1[...]
    c1 = jnp.dot(t1[...], w1[...], preferred_element_type=jnp.float32)
    r = dv * rs
    z = jnp.dot(lhs, w1[...], preferred_element_type=jnp.float32)
    z = jnp.maximum(z + r * c1 + b1[...], 0.0)
    z1_o[...] = z

    @pl.when(i == 0)
    def _():
        asum[...] = jnp.zeros_like(asum)
        asq[...] = jnp.zeros_like(asq)

    row = i * B + lax.broadcasted_iota(jnp.int32, (B, 1), 0)
    zm = jnp.where(row < N, z, 0.0)
    asum[...] += jnp.sum(zm, axis=0, keepdims=True)
    asq[...] += jnp.sum(zm * zm, axis=0, keepdims=True)

    @pl.when(i == pl.num_programs(0) - 1)
    def _():
        mu = asum[...] * NINV
        var = asq[...] * NINV - mu * mu
        s2 = g2[...] * lax.rsqrt(var + EPS)
        s2_o[...] = s2
        t2_o[...] = be2[...] - mu * s2


def _t3_call(g1a, g1b, dinv, w1, s1, t1, b1, g2, be2):
    B = 512
    grid = NP // B
    return pl.pallas_call(
        _t3_body,
        grid=(grid,),
        in_specs=[
            pl.BlockSpec((B, 128), lambda i: (i, 0)),
            pl.BlockSpec((B, 128), lambda i: (i, 0)),
            pl.BlockSpec((B, 1), lambda i: (i, 0)),
            pl.BlockSpec((128, 1024), lambda i: (0, 0)),
            pl.BlockSpec((1, 128), lambda i: (0, 0)),
            pl.BlockSpec((1, 128), lambda i: (0, 0)),
            pl.BlockSpec((1, 1024), lambda i: (0, 0)),
            pl.BlockSpec((1, 1024), lambda i: (0, 0)),
            pl.BlockSpec((1, 1024), lambda i: (0, 0)),
        ],
        out_specs=[
            pl.BlockSpec((B, 1024), lambda i: (i, 0)),
            pl.BlockSpec((B, 1), lambda i: (i, 0)),
            pl.BlockSpec((1, 1024), lambda i: (0, 0)),
            pl.BlockSpec((1, 1024), lambda i: (0, 0)),
        ],
        out_shape=[
            jax.ShapeDtypeStruct((NP, 1024), jnp.float32),
            jax.ShapeDtypeStruct((NP, 1), jnp.float32),
            jax.ShapeDtypeStruct((1, 1024), jnp.float32),
            jax.ShapeDtypeStruct((1, 1024), jnp.float32),
        ],
        scratch_shapes=[
            pltpu.VMEM((1, 1024), jnp.float32),
            pltpu.VMEM((1, 1024), jnp.float32),
        ],
    )(g1a, g1b, dinv, w1, s1, t1, b1, g2, be2)


def _t4_body(z1, s2, t2, dinv, w2, yb_o, c2_o):
    zs = z1[...] * s2[...]
    w = w2[...]
    yb_o[0] = jnp.dot(zs, w, preferred_element_type=jnp.float32) * dinv[...]
    c2_o[...] = jnp.dot(t2[...], w, preferred_element_type=jnp.float32)


def _t4_call(z1, s2, t2, dinv, w2):
    B = 512
    gi = NP // B
    return pl.pallas_call(
        _t4_body,
        grid=(gi, 8),
        in_specs=[
            pl.BlockSpec((B, 1024), lambda i, c: (i, 0)),
            pl.BlockSpec((1, 1024), lambda i, c: (0, 0)),
            pl.BlockSpec((1, 1024), lambda i, c: (0, 0)),
            pl.BlockSpec((B, 1), lambda i, c: (i, 0)),
            pl.BlockSpec((1024, 128), lambda i, c: (0, c)),
        ],
        out_specs=[
            pl.BlockSpec((1, B, 128), lambda i, c: (c, i, 0)),
            pl.BlockSpec((1, 128), lambda i, c: (0, c)),
        ],
        out_shape=[
            jax.ShapeDtypeStruct((8, NP, 128), jnp.float32),
            jax.ShapeDtypeStruct((1, 1024), jnp.float32),
        ],
    )(z1, s2, t2, dinv, w2)


def _t6_body(g2, dinv, rs, c2, b2, g3, be3, pool_o, ssum, ssq, smax):
    i = pl.program_id(0)
    B = dinv.shape[0]
    dv = dinv[...]
    r = dv * rs[...]
    row = i * B + lax.broadcasted_iota(jnp.int32, (B, 1), 0)
    valid = row < N

    @pl.when(i == 0)
    def _():
        ssum[...] = jnp.zeros_like(ssum)
        ssq[...] = jnp.zeros_like(ssq)
        smax[...] = jnp.full_like(smax, -3e38)

    for c in range(8):
        cs = slice(c * 128, (c + 1) * 128)
        z2 = jnp.maximum(dv * g2[c] + r * c2[:, cs] + b2[:, cs], 0.0)
        zm = jnp.where(valid, z2, 0.0)
        ssum[pl.ds(c, 1), :] += jnp.sum(zm, axis=0, keepdims=True)
        ssq[pl.ds(c, 1), :] += jnp.sum(zm * zm, axis=0, keepdims=True)
        zx = jnp.where(valid, z2, -3e38)
        smax[pl.ds(c, 1), :] = jnp.maximum(
            smax[pl.ds(c, 1), :], jnp.max(zx, axis=0, keepdims=True))

    @pl.when(i == pl.num_programs(0) - 1)
    def _():
        for c in range(8):
            cs = slice(c * 128, (c + 1) * 128)
            mu = ssum[pl.ds(c, 1), :] * NINV
            var = ssq[pl.ds(c, 1), :] * NINV - mu * mu
            pool_o[pl.ds(c, 1), :] = (
                (smax[pl.ds(c, 1), :] - mu) * lax.rsqrt(var + EPS)
                * g3[:, cs] + be3[:, cs])


def _t6_call(g2, dinv, rs, c2, b2, g3, be3):
    B = 512
    grid = NP // B
    return pl.pallas_call(
        _t6_body,
        grid=(grid,),
        in_specs=[
            pl.BlockSpec((8, B, 128), lambda i: (0, i, 0)),
            pl.BlockSpec((B, 1), lambda i: (i, 0)),
            pl.BlockSpec((B, 1), lambda i: (i, 0)),
            pl.BlockSpec((1, 1024), lambda i: (0, 0)),
            pl.BlockSpec((1, 1024), lambda i: (0, 0)),
            pl.BlockSpec((1, 1024), lambda i: (0, 0)),
            pl.BlockSpec((1, 1024), lambda i: (0, 0)),
        ],
        out_specs=[pl.BlockSpec((8, 128), lambda i: (0, 0))],
        out_shape=[jax.ShapeDtypeStruct((8, 128), jnp.float32)],
        scratch_shapes=[
            pltpu.VMEM((8, 128), jnp.float32),
            pltpu.VMEM((8, 128), jnp.float32),
            pltpu.VMEM((8, 128), jnp.float32),
        ],
    )(g2, dinv, rs, c2, b2, g3, be3)


def _t7_body(pool, m1, m2, fc1w, fc1b, fc2w, fc2b, out_o):
    h = jnp.dot(pool[...] * m1[...], fc1w[...], preferred_element_type=jnp.float32)
    h = jnp.maximum(h + fc1b[...], 0.0)
    o = jnp.dot(h * m2[...], fc2w[...], preferred_element_type=jnp.float32)
    o = o + fc2b[...]
    out_o[...] = 1.0 / (1.0 + jnp.exp(-o))


def _t7_call(pool, m1, m2, fc1w, fc1b, fc2w, fc2b):
    return pl.pallas_call(
        _t7_body,
        out_shape=jax.ShapeDtypeStruct((1, 128), jnp.float32),
    )(pool, m1, m2, fc1w, fc1b, fc2w, fc2b)


@functools.cache
def _sc_mesh():
    return plsc.VectorSubcoreMesh(core_axis_name="c", subcore_axis_name="s",
                                  num_cores=2, num_subcores=16)


def _lazy_sc_kernel(out_type, scratch_types):
    def deco(body):
        @functools.cache
        def build():
            return pl.kernel(
                body, out_type=out_type, mesh=_sc_mesh(),
                scratch_types=scratch_types,
                compiler_params=pltpu.CompilerParams(
                    needs_layout_passes=False,
                    use_tc_tiling_on_sc=False))

        def call(*args):
            return build()(*args)
        return call
    return deco


def _zero_ref(ref, n):
    z = jnp.zeros((16,), jnp.float32)

    def body(i, _):
        ref[pl.ds(i * 16, 16)] = z
        return 0

    lax.fori_loop(0, n // 16, body, 0, unroll=4)


def _tile_reduce(part_v, spm, out_h, cid, sid, tmp, acc2, sem):
    pltpu.sync_copy(part_v, spm.at[sid])
    plsc.subcore_barrier()
    _zero_ref(acc2, RPT)
    base = sid * RPT
    for p in range(16):
        pltpu.sync_copy(spm.at[p, pl.ds(base, RPT)], tmp)

        def body(j, _):
            acc2[pl.ds(j * 16, 16)] += tmp[pl.ds(j * 16, 16)]
            return 0

        lax.fori_loop(0, RPT // 16, body, 0, unroll=4)
    pltpu.sync_copy(acc2, out_h.at[cid, pl.ds(base, RPT)])


@_lazy_sc_kernel(
    out_type=[jax.ShapeDtypeStruct((EP,), jnp.int32),
              jax.ShapeDtypeStruct((2, NP), jnp.float32)],
    scratch_types=[
        pltpu.VMEM((EW,), jnp.int32),
        pltpu.VMEM((EW,), jnp.int32),
        pltpu.VMEM((EW,), jnp.int32),
        pltpu.VMEM((NP,), jnp.float32),
        pltpu.VMEM_SHARED((16, NP), jnp.float32),
        pltpu.VMEM((RPT,), jnp.float32),
        pltpu.VMEM((RPT,), jnp.float32),
        pltpu.SemaphoreType.DMA,
    ])
def _k1(dst_h, mask_h, dstp_h, degp_h, dstw, maskw, dpw, acc, spm, tmp, acc2, sem):
    cid = lax.axis_index("c")
    sid = lax.axis_index("s")
    wid = sid * 2 + cid
    base = wid * EW
    _zero_ref(acc, NP)
    pltpu.sync_copy(dst_h.at[pl.ds(base, EW)], dstw)
    pltpu.sync_copy(mask_h.at[pl.ds(base, EW)], maskw)
    dummy = N + ((lax.iota(jnp.int32, 16) + wid * 16) & 63)

    def body(j, _):
        dv = dstw[pl.ds(j * 16, 16)]
        mv = maskw[pl.ds(j * 16, 16)]
        dp = jnp.where(mv > 0, dv, dummy)
        dpw[pl.ds(j * 16, 16)] = dp
        cnt, last = plsc.scan_count(dp)
        plsc.addupdate_scatter(acc, [dp], cnt.astype(jnp.float32), mask=last)
        return 0

    lax.fori_loop(0, EW // 16, body, 0, unroll=2)
    pltpu.sync_copy(dpw, dstp_h.at[pl.ds(base, EW)])
    _tile_reduce(acc, spm, degp_h, cid, sid, tmp, acc2, sem)


@_lazy_sc_kernel(
    out_type=jax.ShapeDtypeStruct((2, NP, 128), jnp.float32),
    scratch_types=[
        pltpu.VMEM((EW,), jnp.int32),
        pltpu.VMEM((NWIN, CW), jnp.int32),
        pltpu.VMEM((CW, 128), jnp.float32),
        pltpu.VMEM_SHARED((NP, 128), jnp.float32),
        pltpu.SemaphoreType.DMA,
    ])
def _k2(src_h, dstp2_h, xp_h, zeros_h, g1p_h, srcw, dpw2, rows, gacc, sem):
    cid = lax.axis_index("c")
    sid = lax.axis_index("s")
    wid = sid * 2 + cid
    base = wid * EW
    pltpu.sync_copy(src_h.at[pl.ds(base, EW)], srcw)
    pltpu.sync_copy(dstp2_h.at[pl.ds(wid * NWIN, NWIN)], dpw2)
    rbase = sid * RPT

    @pl.when(cid == 0)
    def _():
        pltpu.sync_copy(xp_h.at[pl.ds(rbase, RPT)], gacc.at[pl.ds(rbase, RPT)])

    @pl.when(cid == 1)
    def _():
        pltpu.sync_copy(zeros_h.at[pl.ds(rbase, RPT)], gacc.at[pl.ds(rbase, RPT)])

    plsc.subcore_barrier()

    def wbody(w, _):
        pltpu.async_copy(xp_h.at[srcw.at[pl.ds(w * CW, CW)]], rows, sem).wait()
        pltpu.sync_copy(rows, gacc.at[dpw2.at[w]], add=True)
        return 0

    lax.fori_loop(0, NWIN, wbody, 0)
    plsc.subcore_barrier()
    pltpu.sync_copy(gacc.at[pl.ds(rbase, RPT)],
                    g1p_h.at[cid, pl.ds(rbase, RPT)])


EW5 = EP // 16
NWIN5 = EW5 // CW


@_lazy_sc_kernel(
    out_type=jax.ShapeDtypeStruct((8, NP, 128), jnp.float32),
    scratch_types=[
        pltpu.VMEM((EW5,), jnp.int32),
        pltpu.VMEM((CW,), jnp.int32),
        pltpu.VMEM((1, CW), jnp.int32),
        pltpu.VMEM((CW, 128), jnp.float32),
        pltpu.VMEM_SHARED((NP, 128), jnp.float32),
        pltpu.SemaphoreType.DMA,
    ])
def _k5(src_h, dstp2_h, yb_h, g2_h, srcw, adjw, dstw, rows, gacc, sem):
    cid = lax.axis_index("c")
    sid = lax.axis_index("s")
    base = sid * EW5
    pltpu.sync_copy(src_h.at[pl.ds(base, EW5)], srcw)
    rbase = sid * RPT
    for k in range(4):
        c = cid * 4 + k
        off = c * NP
        pltpu.sync_copy(yb_h.at[pl.ds(off + rbase, RPT)],
                        gacc.at[pl.ds(rbase, RPT)])
        plsc.subcore_barrier()

        def wbody(w, _):
            pltpu.sync_copy(dstp2_h.at[pl.ds(sid * NWIN5 + w, 1)], dstw)

            def abody(j, _):
                adjw[pl.ds(j * 16, 16)] = srcw[pl.ds(w * CW + j * 16, 16)] + off
                return 0

            lax.fori_loop(0, CW // 16, abody, 0, unroll=4)
            pltpu.async_copy(yb_h.at[adjw.at[pl.ds(0, CW)]], rows, sem).wait()
            pltpu.sync_copy(rows, gacc.at[dstw.at[0]], add=True)
            return 0

        lax.fori_loop(0, NWIN5, wbody, 0)
        plsc.subcore_barrier()
        pltpu.sync_copy(gacc.at[pl.ds(rbase, RPT)],
                        g2_h.at[c, pl.ds(rbase, RPT)])


def kernel(x, edge_index, W1, b1, W2, b2, g1, be1, g2, be2, g3, be3,
           fc1W, fc1b, fc2W, fc2b):
    f32 = jnp.float32
    kdrop = jax.random.key(42)
    ke, k1, k2 = jax.random.split(kdrop, 3)
    mask = jax.random.bernoulli(ke, 0.8, (E,))
    m1 = jax.random.bernoulli(k1, 0.1, (1, 1024)).astype(f32) * 10.0
    m2 = jax.random.bernoulli(k2, 0.1, (1, 1024)).astype(f32) * 10.0

    src = edge_index[0]
    dst = edge_index[1]
    npad = EP - E
    srcp = jnp.concatenate([src, jnp.arange(npad, dtype=jnp.int32)])
    dstpad = jnp.concatenate([dst, jnp.zeros((npad,), jnp.int32)])
    maskp = jnp.concatenate([mask.astype(jnp.int32),
                             jnp.zeros((npad,), jnp.int32)])

    xpad = jnp.pad(x, ((0, NP - N), (0, 28)))
    g1p = jnp.pad(g1, (0, 28)).reshape(1, 128)
    be1p = jnp.pad(be1, (0, 28)).reshape(1, 128)
    w1p = jnp.pad(W1, ((0, 28), (0, 0)))
    b1r = b1.reshape(1, 1024)
    g2r = g2.reshape(1, 1024)
    be2r = be2.reshape(1, 1024)
    b2r = b2.reshape(1, 1024)
    g3r = g3.reshape(1, 1024)
    be3r = be3.reshape(1, 1024)
    fc1br = fc1b.reshape(1, 1024)
    fc2wp = jnp.pad(fc2W, ((0, 0), (0, 28)))
    fc2bp = jnp.pad(fc2b, (0, 28)).reshape(1, 128)

    dstp, degp = _k1(dstpad, maskp)
    deg0 = degp[0].reshape(NP, 1)
    deg1 = degp[1].reshape(NP, 1)

    dinv, xp, s1, t1 = _t1_call(deg0, deg1, xpad, g1p, be1p)
    dstp2 = dstp.reshape(EP // CW, CW)

    zeros_np = jnp.zeros((NP, 128), f32)
    g1p_agg = _k2(srcp, dstp2, xp, zeros_np)
    g1a = g1p_agg[0]
    g1b = g1p_agg[1]

    z1, rs, s2, t2 = _t3_call(g1a, g1b, dinv, w1p, s1, t1, b1r, g2r, be2r)
    yb, c2 = _t4_call(z1, s2, t2, dinv, W2)

    ybflat = yb.reshape(8 * NP, 128)
    g2agg = _k5(srcp, dstp2, ybflat)

    pool, = _t6_call(g2agg, dinv, rs, c2, b2r, g3r, be3r)
    poolr = pool.reshape(1, 1024)

    out = _t7_call(poolr, m1, m2, fc1W, fc1br, fc2wp, fc2bp)
    return out[:, :100]


# speedup vs baseline: 8.8806x; 1.0608x over previous
import functools
import jax
import jax.numpy as jnp
from jax import lax
from jax.experimental import pallas as pl
from jax.experimental.pallas import tpu as pltpu
from jax.experimental.pallas import tpu_sc as plsc

N = 10000
NP = 10240
E = 160000
EPS = 1e-5
NINV = 1.0 / N

EP = 163840
NWK = 32
EW = EP // NWK
CW = 256
NWIN = EW // CW
RPT = NP // 16


def _t1_body(deg0, deg1, xpad, g1, be1, dinv_o, xp_o, s1_o, t1_o, asum, asq):
    i = pl.program_id(0)
    B = xpad.shape[0]
    deg = deg0[...] + deg1[...] + 1.0
    dinv = lax.rsqrt(deg)
    dinv_o[...] = dinv
    x = xpad[...]
    col = lax.broadcasted_iota(jnp.int32, (B, 128), 1)
    xp_o[...] = jnp.where(col == 127, dinv, dinv * x)

    @pl.when(i == 0)
    def _():
        asum[...] = jnp.zeros_like(asum)
        asq[...] = jnp.zeros_like(asq)

    asum[...] += jnp.sum(x, axis=0, keepdims=True)
    asq[...] += jnp.sum(x * x, axis=0, keepdims=True)

    @pl.when(i == pl.num_programs(0) - 1)
    def _():
        mu = asum[...] * NINV
        var = asq[...] * NINV - mu * mu
        s1 = g1[...] * lax.rsqrt(var + EPS)
        s1_o[...] = s1
        t1_o[...] = be1[...] - mu * s1


def _t1_call(deg0, deg1, xpad, g1, be1):
    B = 512
    grid = NP // B
    return pl.pallas_call(
        _t1_body,
        grid=(grid,),
        in_specs=[
            pl.BlockSpec((B, 1), lambda i: (i, 0)),
            pl.BlockSpec((B, 1), lambda i: (i, 0)),
            pl.BlockSpec((B, 128), lambda i: (i, 0)),
            pl.BlockSpec((1, 128), lambda i: (0, 0)),
            pl.BlockSpec((1, 128), lambda i: (0, 0)),
        ],
        out_specs=[
            pl.BlockSpec((B, 1), lambda i: (i, 0)),
            pl.BlockSpec((B, 128), lambda i: (i, 0)),
            pl.BlockSpec((1, 128), lambda i: (0, 0)),
            pl.BlockSpec((1, 128), lambda i: (0, 0)),
        ],
        out_shape=[
            jax.ShapeDtypeStruct((NP, 1), jnp.float32),
            jax.ShapeDtypeStruct((NP, 128), jnp.float32),
            jax.ShapeDtypeStruct((1, 128), jnp.float32),
            jax.ShapeDtypeStruct((1, 128), jnp.float32),
        ],
        scratch_shapes=[
            pltpu.VMEM((1, 128), jnp.float32),
            pltpu.VMEM((1, 128), jnp.float32),
        ],
    )(deg0, deg1, xpad, g1, be1)


def _t3_body(g1a, g1b, dinv, w1, s1, t1, b1, g2, be2,
             z1_o, rs_o, s2_o, t2_o, asum, asq):
    i = pl.program_id(0)
    B = g1a.shape[0]
    dv = dinv[...]
    g = g1a[...] + g1b[...]
    rs = g[:, 127:128]
    rs_o[...] = rs
    lhs = g * dv * s1[...]
    c1 = jnp.dot(t1[...], w1[...], preferred_element_type=jnp.float32)
    r = dv * rs
    z = jnp.dot(lhs, w1[...], preferred_element_type=jnp.float32)
    z = jnp.maximum(z + r * c1 + b1[...], 0.0)
    z1_o[...] = z

    @pl.when(i == 0)
    def _():
        asum[...] = jnp.zeros_like(asum)
        asq[...] = jnp.zeros_like(asq)

    row = i * B + lax.broadcasted_iota(jnp.int32, (B, 1), 0)
    zm = jnp.where(row < N, z, 0.0)
    asum[...] += jnp.sum(zm, axis=0, keepdims=True)
    asq[...] += jnp.sum(zm * zm, axis=0, keepdims=True)

    @pl.when(i == pl.num_programs(0) - 1)
    def _():
        mu = asum[...] * NINV
        var = asq[...] * NINV - mu * mu
        s2 = g2[...] * lax.rsqrt(var + EPS)
        s2_o[...] = s2
        t2_o[...] = be2[...] - mu * s2


def _t3_call(g1a, g1b, dinv, w1, s1, t1, b1, g2, be2):
    B = 512
    grid = NP // B
    return pl.pallas_call(
        _t3_body,
        grid=(grid,),
        in_specs=[
            pl.BlockSpec((B, 128), lambda i: (i, 0)),
            pl.BlockSpec((B, 128), lambda i: (i, 0)),
            pl.BlockSpec((B, 1), lambda i: (i, 0)),
            pl.BlockSpec((128, 1024), lambda i: (0, 0)),
            pl.BlockSpec((1, 128), lambda i: (0, 0)),
            pl.BlockSpec((1, 128), lambda i: (0, 0)),
            pl.BlockSpec((1, 1024), lambda i: (0, 0)),
            pl.BlockSpec((1, 1024), lambda i: (0, 0)),
            pl.BlockSpec((1, 1024), lambda i: (0, 0)),
        ],
        out_specs=[
            pl.BlockSpec((B, 1024), lambda i: (i, 0)),
            pl.BlockSpec((B, 1), lambda i: (i, 0)),
            pl.BlockSpec((1, 1024), lambda i: (0, 0)),
            pl.BlockSpec((1, 1024), lambda i: (0, 0)),
        ],
        out_shape=[
            jax.ShapeDtypeStruct((NP, 1024), jnp.float32),
            jax.ShapeDtypeStruct((NP, 1), jnp.float32),
            jax.ShapeDtypeStruct((1, 1024), jnp.float32),
            jax.ShapeDtypeStruct((1, 1024), jnp.float32),
        ],
        scratch_shapes=[
            pltpu.VMEM((1, 1024), jnp.float32),
            pltpu.VMEM((1, 1024), jnp.float32),
        ],
    )(g1a, g1b, dinv, w1, s1, t1, b1, g2, be2)


def _t4_body(z1, s2, t2, dinv, w2, yb_o, c2_o):
    zs = z1[...] * s2[...]
    w = w2[...]
    yb_o[0] = jnp.dot(zs, w, preferred_element_type=jnp.float32) * dinv[...]
    c2_o[0] = jnp.dot(t2[...], w, preferred_element_type=jnp.float32)


def _t4_call(z1, s2, t2, dinv, w2, j):
    B = 512
    gi = NP // B
    return pl.pallas_call(
        _t4_body,
        grid=(gi, 2),
        in_specs=[
            pl.BlockSpec((B, 1024), lambda i, s: (i, 0)),
            pl.BlockSpec((1, 1024), lambda i, s: (0, 0)),
            pl.BlockSpec((1, 1024), lambda i, s: (0, 0)),
            pl.BlockSpec((B, 1), lambda i, s: (i, 0)),
            pl.BlockSpec((1024, 128), lambda i, s: (0, j + 4 * s)),
        ],
        out_specs=[
            pl.BlockSpec((1, B, 128), lambda i, s: (s, i, 0)),
            pl.BlockSpec((1, 1, 128), lambda i, s: (s, 0, 0)),
        ],
        out_shape=[
            jax.ShapeDtypeStruct((2, NP, 128), jnp.float32),
            jax.ShapeDtypeStruct((2, 1, 128), jnp.float32),
        ],
    )(z1, s2, t2, dinv, w2)


def _t6_body(g2, dinv, rs, c2, b2, g3, be3, pool_o, ssum, ssq, smax):
    i = pl.program_id(0)
    B = dinv.shape[0]
    dv = dinv[...]
    r = dv * rs[...]
    row = i * B + lax.broadcasted_iota(jnp.int32, (B, 1), 0)
    valid = row < N

    @pl.when(i == 0)
    def _():
        ssum[...] = jnp.zeros_like(ssum)
        ssq[...] = jnp.zeros_like(ssq)
        smax[...] = jnp.full_like(smax, -3e38)

    for s in range(2):
        z2 = jnp.maximum(dv * g2[s] + r * c2[s] + b2[s], 0.0)
        zm = jnp.where(valid, z2, 0.0)
        ssum[pl.ds(s, 1), :] += jnp.sum(zm, axis=0, keepdims=True)
        ssq[pl.ds(s, 1), :] += jnp.sum(zm * zm, axis=0, keepdims=True)
        zx = jnp.where(valid, z2, -3e38)
        smax[pl.ds(s, 1), :] = jnp.maximum(
            smax[pl.ds(s, 1), :], jnp.max(zx, axis=0, keepdims=True))

    @pl.when(i == pl.num_programs(0) - 1)
    def _():
        for s in range(2):
            mu = ssum[pl.ds(s, 1), :] * NINV
            var = ssq[pl.ds(s, 1), :] * NINV - mu * mu
            pool_o[s] = ((smax[pl.ds(s, 1), :] - mu) * lax.rsqrt(var + EPS)
                         * g3[s] + be3[s])


def _t6_call(g2, dinv, rs, c2, b2, g3, be3, j):
    B = 512
    grid = NP // B
    return pl.pallas_call(
        _t6_body,
        grid=(grid,),
        in_specs=[
            pl.BlockSpec((2, B, 128), lambda i: (0, i, 0)),
            pl.BlockSpec((B, 1), lambda i: (i, 0)),
            pl.BlockSpec((B, 1), lambda i: (i, 0)),
            pl.BlockSpec((2, 1, 128), lambda i: (0, 0, 0)),
            pl.BlockSpec((2, 1, 128), lambda i: (0, 0, 0)),
            pl.BlockSpec((2, 1, 128), lambda i: (0, 0, 0)),
            pl.BlockSpec((2, 1, 128), lambda i: (0, 0, 0)),
        ],
        out_specs=[pl.BlockSpec((2, 1, 128), lambda i: (0, 0, 0))],
        out_shape=[jax.ShapeDtypeStruct((2, 1, 128), jnp.float32)],
        scratch_shapes=[
            pltpu.VMEM((2, 128), jnp.float32),
            pltpu.VMEM((2, 128), jnp.float32),
            pltpu.VMEM((2, 128), jnp.float32),
        ],
    )(g2, dinv, rs, c2, b2, g3, be3)


def _t7_body(pool, m1, m2, fc1w, fc1b, fc2w, fc2b, out_o):
    h = jnp.dot(pool[...] * m1[...], fc1w[...], preferred_element_type=jnp.float32)
    h = jnp.maximum(h + fc1b[...], 0.0)
    o = jnp.dot(h * m2[...], fc2w[...], preferred_element_type=jnp.float32)
    o = o + fc2b[...]
    out_o[...] = 1.0 / (1.0 + jnp.exp(-o))


def _t7_call(pool, m1, m2, fc1w, fc1b, fc2w, fc2b):
    return pl.pallas_call(
        _t7_body,
        out_shape=jax.ShapeDtypeStruct((1, 128), jnp.float32),
    )(pool, m1, m2, fc1w, fc1b, fc2w, fc2b)


@functools.cache
def _sc_mesh():
    return plsc.VectorSubcoreMesh(core_axis_name="c", subcore_axis_name="s",
                                  num_cores=2, num_subcores=16)


def _lazy_sc_kernel(out_type, scratch_types):
    def deco(body):
        @functools.cache
        def build():
            return pl.kernel(
                body, out_type=out_type, mesh=_sc_mesh(),
                scratch_types=scratch_types,
                compiler_params=pltpu.CompilerParams(
                    needs_layout_passes=False,
                    use_tc_tiling_on_sc=False))

        def call(*args):
            return build()(*args)
        return call
    return deco


def _zero_ref(ref, n):
    z = jnp.zeros((16,), jnp.float32)

    def body(i, _):
        ref[pl.ds(i * 16, 16)] = z
        return 0

    lax.fori_loop(0, n // 16, body, 0, unroll=4)


def _tile_reduce(part_v, spm, out_h, cid, sid, tmp, acc2, sem):
    pltpu.sync_copy(part_v, spm.at[sid])
    plsc.subcore_barrier()
    _zero_ref(acc2, RPT)
    base = sid * RPT
    for p in range(16):
        pltpu.sync_copy(spm.at[p, pl.ds(base, RPT)], tmp)

        def body(j, _):
            acc2[pl.ds(j * 16, 16)] += tmp[pl.ds(j * 16, 16)]
            return 0

        lax.fori_loop(0, RPT // 16, body, 0, unroll=4)
    pltpu.sync_copy(acc2, out_h.at[cid, pl.ds(base, RPT)])


@_lazy_sc_kernel(
    out_type=[jax.ShapeDtypeStruct((EP,), jnp.int32),
              jax.ShapeDtypeStruct((2, NP), jnp.float32)],
    scratch_types=[
        pltpu.VMEM((EW,), jnp.int32),
        pltpu.VMEM((EW,), jnp.int32),
        pltpu.VMEM((EW,), jnp.int32),
        pltpu.VMEM((NP,), jnp.float32),
        pltpu.VMEM_SHARED((16, NP), jnp.float32),
        pltpu.VMEM((RPT,), jnp.float32),
        pltpu.VMEM((RPT,), jnp.float32),
        pltpu.SemaphoreType.DMA,
    ])
def _k1(dst_h, mask_h, dstp_h, degp_h, dstw, maskw, dpw, acc, spm, tmp, acc2, sem):
    cid = lax.axis_index("c")
    sid = lax.axis_index("s")
    wid = sid * 2 + cid
    base = wid * EW
    _zero_ref(acc, NP)
    pltpu.sync_copy(dst_h.at[pl.ds(base, EW)], dstw)
    pltpu.sync_copy(mask_h.at[pl.ds(base, EW)], maskw)
    dummy = N + ((lax.iota(jnp.int32, 16) + wid * 16) & 63)

    def body(j, _):
        dv = dstw[pl.ds(j * 16, 16)]
        mv = maskw[pl.ds(j * 16, 16)]
        dp = jnp.where(mv > 0, dv, dummy)
        dpw[pl.ds(j * 16, 16)] = dp
        cnt, last = plsc.scan_count(dp)
        plsc.addupdate_scatter(acc, [dp], cnt.astype(jnp.float32), mask=last)
        return 0

    lax.fori_loop(0, EW // 16, body, 0, unroll=2)
    pltpu.sync_copy(dpw, dstp_h.at[pl.ds(base, EW)])
    _tile_reduce(acc, spm, degp_h, cid, sid, tmp, acc2, sem)


@_lazy_sc_kernel(
    out_type=jax.ShapeDtypeStruct((2, NP, 128), jnp.float32),
    scratch_types=[
        pltpu.VMEM((EW,), jnp.int32),
        pltpu.VMEM((NWIN, CW), jnp.int32),
        pltpu.VMEM((CW, 128), jnp.float32),
        pltpu.VMEM_SHARED((NP, 128), jnp.float32),
        pltpu.SemaphoreType.DMA,
    ])
def _k2(src_h, dstp2_h, xp_h, zeros_h, g1p_h, srcw, dpw2, rows, gacc, sem):
    cid = lax.axis_index("c")
    sid = lax.axis_index("s")
    wid = sid * 2 + cid
    base = wid * EW
    pltpu.sync_copy(src_h.at[pl.ds(base, EW)], srcw)
    pltpu.sync_copy(dstp2_h.at[pl.ds(wid * NWIN, NWIN)], dpw2)
    rbase = sid * RPT

    @pl.when(cid == 0)
    def _():
        pltpu.sync_copy(xp_h.at[pl.ds(rbase, RPT)], gacc.at[pl.ds(rbase, RPT)])

    @pl.when(cid == 1)
    def _():
        pltpu.sync_copy(zeros_h.at[pl.ds(rbase, RPT)], gacc.at[pl.ds(rbase, RPT)])

    plsc.subcore_barrier()

    def wbody(w, _):
        pltpu.async_copy(xp_h.at[srcw.at[pl.ds(w * CW, CW)]], rows, sem).wait()
        pltpu.sync_copy(rows, gacc.at[dpw2.at[w]], add=True)
        return 0

    lax.fori_loop(0, NWIN, wbody, 0)
    plsc.subcore_barrier()
    pltpu.sync_copy(gacc.at[pl.ds(rbase, RPT)],
                    g1p_h.at[cid, pl.ds(rbase, RPT)])


EW5 = EP // 16
NWIN5 = EW5 // CW


@_lazy_sc_kernel(
    out_type=jax.ShapeDtypeStruct((2, NP, 128), jnp.float32),
    scratch_types=[
        pltpu.VMEM((EW5,), jnp.int32),
        pltpu.VMEM((CW,), jnp.int32),
        pltpu.VMEM((1, CW), jnp.int32),
        pltpu.VMEM((CW, 128), jnp.float32),
        pltpu.VMEM_SHARED((NP, 128), jnp.float32),
        pltpu.SemaphoreType.DMA,
    ])
def _k5(src_h, dstp2_h, yb_h, g2_h, srcw, adjw, dstw, rows, gacc, sem):
    cid = lax.axis_index("c")
    sid = lax.axis_index("s")
    base = sid * EW5
    pltpu.sync_copy(src_h.at[pl.ds(base, EW5)], srcw)
    rbase = sid * RPT
    off = cid * NP
    pltpu.sync_copy(yb_h.at[pl.ds(off + rbase, RPT)],
                    gacc.at[pl.ds(rbase, RPT)])
    plsc.subcore_barrier()

    def wbody(w, _):
        pltpu.sync_copy(dstp2_h.at[pl.ds(sid * NWIN5 + w, 1)], dstw)

        def abody(j, _):
            adjw[pl.ds(j * 16, 16)] = srcw[pl.ds(w * CW + j * 16, 16)] + off
            return 0

        lax.fori_loop(0, CW // 16, abody, 0, unroll=4)
        pltpu.async_copy(yb_h.at[adjw.at[pl.ds(0, CW)]], rows, sem).wait()
        pltpu.sync_copy(rows, gacc.at[dstw.at[0]], add=True)
        return 0

    lax.fori_loop(0, NWIN5, wbody, 0)
    plsc.subcore_barrier()
    pltpu.sync_copy(gacc.at[pl.ds(rbase, RPT)],
                    g2_h.at[cid, pl.ds(rbase, RPT)])


def kernel(x, edge_index, W1, b1, W2, b2, g1, be1, g2, be2, g3, be3,
           fc1W, fc1b, fc2W, fc2b):
    f32 = jnp.float32
    kdrop = jax.random.key(42)
    ke, k1, k2 = jax.random.split(kdrop, 3)
    mask = jax.random.bernoulli(ke, 0.8, (E,))
    m1 = jax.random.bernoulli(k1, 0.1, (1, 1024)).astype(f32) * 10.0
    m2 = jax.random.bernoulli(k2, 0.1, (1, 1024)).astype(f32) * 10.0

    src = edge_index[0]
    dst = edge_index[1]
    npad = EP - E
    srcp = jnp.concatenate([src, jnp.arange(npad, dtype=jnp.int32)])
    dstpad = jnp.concatenate([dst, jnp.zeros((npad,), jnp.int32)])
    maskp = jnp.concatenate([mask.astype(jnp.int32),
                             jnp.zeros((npad,), jnp.int32)])

    xpad = jnp.pad(x, ((0, NP - N), (0, 28)))
    g1p = jnp.pad(g1, (0, 28)).reshape(1, 128)
    be1p = jnp.pad(be1, (0, 28)).reshape(1, 128)
    w1p = jnp.pad(W1, ((0, 28), (0, 0)))
    b1r = b1.reshape(1, 1024)
    g2r = g2.reshape(1, 1024)
    be2r = be2.reshape(1, 1024)
    b2r = b2.reshape(1, 1024)
    g3r = g3.reshape(1, 1024)
    be3r = be3.reshape(1, 1024)
    fc1br = fc1b.reshape(1, 1024)
    fc2wp = jnp.pad(fc2W, ((0, 0), (0, 28)))
    fc2bp = jnp.pad(fc2b, (0, 28)).reshape(1, 128)

    dstp, degp = _k1(dstpad, maskp)
    deg0 = degp[0].reshape(NP, 1)
    deg1 = degp[1].reshape(NP, 1)

    dinv, xp, s1, t1 = _t1_call(deg0, deg1, xpad, g1p, be1p)
    dstp2 = dstp.reshape(EP // CW, CW)

    zeros_np = jnp.zeros((NP, 128), f32)
    g1p_agg = _k2(srcp, dstp2, xp, zeros_np)
    g1a = g1p_agg[0]
    g1b = g1p_agg[1]

    z1, rs, s2, t2 = _t3_call(g1a, g1b, dinv, w1p, s1, t1, b1r, g2r, be2r)

    def strips(v):
        m = v.reshape(8, 1, 128)
        return [jnp.stack([m[j], m[j + 4]]) for j in range(4)]

    b2s = strips(b2)
    g3s = strips(g3)
    be3s = strips(be3)
    pools = []
    for j in range(4):
        yb_j, c2_j = _t4_call(z1, s2, t2, dinv, W2, j)
        g2_j = _k5(srcp, dstp2, yb_j.reshape(2 * NP, 128))
        pool_j, = _t6_call(g2_j, dinv, rs, c2_j, b2s[j], g3s[j], be3s[j], j)
        pools.append(pool_j)

    pool8 = jnp.concatenate(
        [jnp.concatenate([pools[j][s] for j in range(4)], axis=0)
         for s in range(2)], axis=0)
    poolr = pool8.reshape(1, 1024)

    out = _t7_call(poolr, m1, m2, fc1W, fc1br, fc2wp, fc2bp)
    return out[:, :100]


# speedup vs baseline: 10.8305x; 1.2196x over previous
import functools
import jax
import jax.numpy as jnp
from jax import lax
from jax.experimental import pallas as pl
from jax.experimental.pallas import tpu as pltpu
from jax.experimental.pallas import tpu_sc as plsc

N = 10000
NP = 10240
E = 160000
EPS = 1e-5
NINV = 1.0 / N

EP = 163840
NWK = 32
EW = EP // NWK
CW = 256
NWIN = EW // CW
RPT = NP // 16


def _t1_body(deg0, deg1, xpad, g1, be1, dinv_o, xp_o, s1_o, t1_o, asum, asq):
    i = pl.program_id(0)
    B = xpad.shape[0]
    deg = deg0[...] + deg1[...] + 1.0
    dinv = lax.rsqrt(deg)
    dinv_o[...] = dinv
    x = xpad[...]
    col = lax.broadcasted_iota(jnp.int32, (B, 128), 1)
    xp_o[...] = jnp.where(col == 127, dinv, dinv * x)

    @pl.when(i == 0)
    def _():
        asum[...] = jnp.zeros_like(asum)
        asq[...] = jnp.zeros_like(asq)

    asum[...] += jnp.sum(x, axis=0, keepdims=True)
    asq[...] += jnp.sum(x * x, axis=0, keepdims=True)

    @pl.when(i == pl.num_programs(0) - 1)
    def _():
        mu = asum[...] * NINV
        var = asq[...] * NINV - mu * mu
        s1 = g1[...] * lax.rsqrt(var + EPS)
        s1_o[...] = s1
        t1_o[...] = be1[...] - mu * s1


def _t1_call(deg0, deg1, xpad, g1, be1):
    B = 512
    grid = NP // B
    return pl.pallas_call(
        _t1_body,
        grid=(grid,),
        in_specs=[
            pl.BlockSpec((B, 1), lambda i: (i, 0)),
            pl.BlockSpec((B, 1), lambda i: (i, 0)),
            pl.BlockSpec((B, 128), lambda i: (i, 0)),
            pl.BlockSpec((1, 128), lambda i: (0, 0)),
            pl.BlockSpec((1, 128), lambda i: (0, 0)),
        ],
        out_specs=[
            pl.BlockSpec((B, 1), lambda i: (i, 0)),
            pl.BlockSpec((B, 128), lambda i: (i, 0)),
            pl.BlockSpec((1, 128), lambda i: (0, 0)),
            pl.BlockSpec((1, 128), lambda i: (0, 0)),
        ],
        out_shape=[
            jax.ShapeDtypeStruct((NP, 1), jnp.float32),
            jax.ShapeDtypeStruct((NP, 128), jnp.float32),
            jax.ShapeDtypeStruct((1, 128), jnp.float32),
            jax.ShapeDtypeStruct((1, 128), jnp.float32),
        ],
        scratch_shapes=[
            pltpu.VMEM((1, 128), jnp.float32),
            pltpu.VMEM((1, 128), jnp.float32),
        ],
    )(deg0, deg1, xpad, g1, be1)


def _t3_body(g1a, g1b, dinv, w1, s1, t1, b1, g2, be2,
             z1_o, rs_o, s2_o, t2_o, asum, asq):
    i = pl.program_id(0)
    B = g1a.shape[0]
    dv = dinv[...]
    g = g1a[...] + g1b[...]
    rs = g[:, 127:128]
    rs_o[...] = rs
    lhs = g * dv * s1[...]
    c1 = jnp.dot(t1[...], w1[...], preferred_element_type=jnp.float32)
    r = dv * rs
    z = jnp.dot(lhs, w1[...], preferred_element_type=jnp.float32)
    z = jnp.maximum(z + r * c1 + b1[...], 0.0)
    z1_o[...] = z

    @pl.when(i == 0)
    def _():
        asum[...] = jnp.zeros_like(asum)
        asq[...] = jnp.zeros_like(asq)

    row = i * B + lax.broadcasted_iota(jnp.int32, (B, 1), 0)
    zm = jnp.where(row < N, z, 0.0)
    asum[...] += jnp.sum(zm, axis=0, keepdims=True)
    asq[...] += jnp.sum(zm * zm, axis=0, keepdims=True)

    @pl.when(i == pl.num_programs(0) - 1)
    def _():
        mu = asum[...] * NINV
        var = asq[...] * NINV - mu * mu
        s2 = g2[...] * lax.rsqrt(var + EPS)
        s2_o[...] = s2
        t2_o[...] = be2[...] - mu * s2


def _t3_call(g1a, g1b, dinv, w1, s1, t1, b1, g2, be2):
    B = 512
    grid = NP // B
    return pl.pallas_call(
        _t3_body,
        grid=(grid,),
        in_specs=[
            pl.BlockSpec((B, 128), lambda i: (i, 0)),
            pl.BlockSpec((B, 128), lambda i: (i, 0)),
            pl.BlockSpec((B, 1), lambda i: (i, 0)),
            pl.BlockSpec((128, 1024), lambda i: (0, 0)),
            pl.BlockSpec((1, 128), lambda i: (0, 0)),
            pl.BlockSpec((1, 128), lambda i: (0, 0)),
            pl.BlockSpec((1, 1024), lambda i: (0, 0)),
            pl.BlockSpec((1, 1024), lambda i: (0, 0)),
            pl.BlockSpec((1, 1024), lambda i: (0, 0)),
        ],
        out_specs=[
            pl.BlockSpec((B, 1024), lambda i: (i, 0)),
            pl.BlockSpec((B, 1), lambda i: (i, 0)),
            pl.BlockSpec((1, 1024), lambda i: (0, 0)),
            pl.BlockSpec((1, 1024), lambda i: (0, 0)),
        ],
        out_shape=[
            jax.ShapeDtypeStruct((NP, 1024), jnp.float32),
            jax.ShapeDtypeStruct((NP, 1), jnp.float32),
            jax.ShapeDtypeStruct((1, 1024), jnp.float32),
            jax.ShapeDtypeStruct((1, 1024), jnp.float32),
        ],
        scratch_shapes=[
            pltpu.VMEM((1, 1024), jnp.float32),
            pltpu.VMEM((1, 1024), jnp.float32),
        ],
    )(g1a, g1b, dinv, w1, s1, t1, b1, g2, be2)


def _t4_body(z1, s2, t2, dinv, w2, yb_o, c2_o):
    zs = z1[...] * s2[...]
    w = w2[...]
    yb_o[0] = jnp.dot(zs, w, preferred_element_type=jnp.float32) * dinv[...]
    c2_o[0] = jnp.dot(t2[...], w, preferred_element_type=jnp.float32)


def _t4_call(z1, s2, t2, dinv, w2, j):
    B = 512
    gi = NP // B
    return pl.pallas_call(
        _t4_body,
        grid=(gi, 2),
        in_specs=[
            pl.BlockSpec((B, 1024), lambda i, s: (i, 0)),
            pl.BlockSpec((1, 1024), lambda i, s: (0, 0)),
            pl.BlockSpec((1, 1024), lambda i, s: (0, 0)),
            pl.BlockSpec((B, 1), lambda i, s: (i, 0)),
            pl.BlockSpec((1024, 128), lambda i, s: (0, j + 4 * s)),
        ],
        out_specs=[
            pl.BlockSpec((1, B, 128), lambda i, s: (s, i, 0)),
            pl.BlockSpec((1, 1, 128), lambda i, s: (s, 0, 0)),
        ],
        out_shape=[
            jax.ShapeDtypeStruct((2, NP, 128), jnp.float32),
            jax.ShapeDtypeStruct((2, 1, 128), jnp.float32),
        ],
    )(z1, s2, t2, dinv, w2)


def _t6_body(g2, dinv, rs, c2, b2, g3, be3, pool_o, ssum, ssq, smax):
    i = pl.program_id(0)
    B = dinv.shape[0]
    dv = dinv[...]
    r = dv * rs[...]
    row = i * B + lax.broadcasted_iota(jnp.int32, (B, 1), 0)
    valid = row < N

    @pl.when(i == 0)
    def _():
        ssum[...] = jnp.zeros_like(ssum)
        ssq[...] = jnp.zeros_like(ssq)
        smax[...] = jnp.full_like(smax, -3e38)

    for s in range(2):
        z2 = jnp.maximum(dv * g2[s] + r * c2[s] + b2[s], 0.0)
        zm = jnp.where(valid, z2, 0.0)
        ssum[pl.ds(s, 1), :] += jnp.sum(zm, axis=0, keepdims=True)
        ssq[pl.ds(s, 1), :] += jnp.sum(zm * zm, axis=0, keepdims=True)
        zx = jnp.where(valid, z2, -3e38)
        smax[pl.ds(s, 1), :] = jnp.maximum(
            smax[pl.ds(s, 1), :], jnp.max(zx, axis=0, keepdims=True))

    @pl.when(i == pl.num_programs(0) - 1)
    def _():
        for s in range(2):
            mu = ssum[pl.ds(s, 1), :] * NINV
            var = ssq[pl.ds(s, 1), :] * NINV - mu * mu
            pool_o[s] = ((smax[pl.ds(s, 1), :] - mu) * lax.rsqrt(var + EPS)
                         * g3[s] + be3[s])


def _t6_call(g2, dinv, rs, c2, b2, g3, be3, j):
    B = 512
    grid = NP // B
    return pl.pallas_call(
        _t6_body,
        grid=(grid,),
        in_specs=[
            pl.BlockSpec((2, B, 128), lambda i: (0, i, 0)),
            pl.BlockSpec((B, 1), lambda i: (i, 0)),
            pl.BlockSpec((B, 1), lambda i: (i, 0)),
            pl.BlockSpec((2, 1, 128), lambda i: (0, 0, 0)),
            pl.BlockSpec((2, 1, 128), lambda i: (0, 0, 0)),
            pl.BlockSpec((2, 1, 128), lambda i: (0, 0, 0)),
            pl.BlockSpec((2, 1, 128), lambda i: (0, 0, 0)),
        ],
        out_specs=[pl.BlockSpec((2, 1, 128), lambda i: (0, 0, 0))],
        out_shape=[jax.ShapeDtypeStruct((2, 1, 128), jnp.float32)],
        scratch_shapes=[
            pltpu.VMEM((2, 128), jnp.float32),
            pltpu.VMEM((2, 128), jnp.float32),
            pltpu.VMEM((2, 128), jnp.float32),
        ],
    )(g2, dinv, rs, c2, b2, g3, be3)


def _t7_body(pool, m1, m2, fc1w, fc1b, fc2w, fc2b, out_o):
    h = jnp.dot(pool[...] * m1[...], fc1w[...], preferred_element_type=jnp.float32)
    h = jnp.maximum(h + fc1b[...], 0.0)
    o = jnp.dot(h * m2[...], fc2w[...], preferred_element_type=jnp.float32)
    o = o + fc2b[...]
    out_o[...] = 1.0 / (1.0 + jnp.exp(-o))


def _t7_call(pool, m1, m2, fc1w, fc1b, fc2w, fc2b):
    return pl.pallas_call(
        _t7_body,
        out_shape=jax.ShapeDtypeStruct((1, 128), jnp.float32),
    )(pool, m1, m2, fc1w, fc1b, fc2w, fc2b)


@functools.cache
def _sc_mesh():
    return plsc.VectorSubcoreMesh(core_axis_name="c", subcore_axis_name="s",
                                  num_cores=2, num_subcores=16)


def _lazy_sc_kernel(out_type, scratch_types):
    def deco(body):
        @functools.cache
        def build():
            return pl.kernel(
                body, out_type=out_type, mesh=_sc_mesh(),
                scratch_types=scratch_types,
                compiler_params=pltpu.CompilerParams(
                    needs_layout_passes=False,
                    use_tc_tiling_on_sc=False))

        def call(*args):
            return build()(*args)
        return call
    return deco


def _zero_ref(ref, n):
    z = jnp.zeros((16,), jnp.float32)

    def body(i, _):
        ref[pl.ds(i * 16, 16)] = z
        return 0

    lax.fori_loop(0, n // 16, body, 0, unroll=4)


def _tile_reduce(part_v, spm, out_h, cid, sid, tmp, acc2, sem):
    pltpu.sync_copy(part_v, spm.at[sid])
    plsc.subcore_barrier()
    _zero_ref(acc2, RPT)
    base = sid * RPT
    for p in range(16):
        pltpu.sync_copy(spm.at[p, pl.ds(base, RPT)], tmp)

        def body(j, _):
            acc2[pl.ds(j * 16, 16)] += tmp[pl.ds(j * 16, 16)]
            return 0

        lax.fori_loop(0, RPT // 16, body, 0, unroll=4)
    pltpu.sync_copy(acc2, out_h.at[cid, pl.ds(base, RPT)])


@_lazy_sc_kernel(
    out_type=[jax.ShapeDtypeStruct((EP,), jnp.int32),
              jax.ShapeDtypeStruct((2, NP), jnp.float32)],
    scratch_types=[
        pltpu.VMEM((EW,), jnp.int32),
        pltpu.VMEM((EW,), jnp.int32),
        pltpu.VMEM((EW,), jnp.int32),
        pltpu.VMEM((NP,), jnp.float32),
        pltpu.VMEM_SHARED((16, NP), jnp.float32),
        pltpu.VMEM((RPT,), jnp.float32),
        pltpu.VMEM((RPT,), jnp.float32),
        pltpu.SemaphoreType.DMA,
    ])
def _k1(dst_h, mask_h, dstp_h, degp_h, dstw, maskw, dpw, acc, spm, tmp, acc2, sem):
    cid = lax.axis_index("c")
    sid = lax.axis_index("s")
    wid = sid * 2 + cid
    base = wid * EW
    _zero_ref(acc, NP)
    pltpu.sync_copy(dst_h.at[pl.ds(base, EW)], dstw)
    pltpu.sync_copy(mask_h.at[pl.ds(base, EW)], maskw)
    dummy = N + ((lax.iota(jnp.int32, 16) + wid * 16) & 63)

    def body(j, _):
        dv = dstw[pl.ds(j * 16, 16)]
        mv = maskw[pl.ds(j * 16, 16)]
        dp = jnp.where(mv > 0, dv, dummy)
        dpw[pl.ds(j * 16, 16)] = dp
        cnt, last = plsc.scan_count(dp)
        plsc.addupdate_scatter(acc, [dp], cnt.astype(jnp.float32), mask=last)
        return 0

    lax.fori_loop(0, EW // 16, body, 0, unroll=2)
    pltpu.sync_copy(dpw, dstp_h.at[pl.ds(base, EW)])
    _tile_reduce(acc, spm, degp_h, cid, sid, tmp, acc2, sem)


@_lazy_sc_kernel(
    out_type=jax.ShapeDtypeStruct((2, NP, 128), jnp.float32),
    scratch_types=[
        pltpu.VMEM((EW,), jnp.int32),
        pltpu.VMEM((NWIN, CW), jnp.int32),
        pltpu.VMEM((CW, 128), jnp.float32),
        pltpu.VMEM_SHARED((NP, 128), jnp.float32),
        pltpu.SemaphoreType.DMA,
    ])
def _k2(src_h, dstp2_h, xp_h, zeros_h, g1p_h, srcw, dpw2, rows, gacc, sem):
    cid = lax.axis_index("c")
    sid = lax.axis_index("s")
    wid = sid * 2 + cid
    base = wid * EW
    pltpu.sync_copy(src_h.at[pl.ds(base, EW)], srcw)
    pltpu.sync_copy(dstp2_h.at[pl.ds(wid * NWIN, NWIN)], dpw2)
    rbase = sid * RPT

    @pl.when(cid == 0)
    def _():
        pltpu.sync_copy(xp_h.at[pl.ds(rbase, RPT)], gacc.at[pl.ds(rbase, RPT)])

    @pl.when(cid == 1)
    def _():
        pltpu.sync_copy(zeros_h.at[pl.ds(rbase, RPT)], gacc.at[pl.ds(rbase, RPT)])

    plsc.subcore_barrier()

    def wbody(w, _):
        pltpu.async_copy(xp_h.at[srcw.at[pl.ds(w * CW, CW)]], rows, sem).wait()
        pltpu.sync_copy(rows, gacc.at[dpw2.at[w]], add=True)
        return 0

    lax.fori_loop(0, NWIN, wbody, 0)
    plsc.subcore_barrier()
    pltpu.sync_copy(gacc.at[pl.ds(rbase, RPT)],
                    g1p_h.at[cid, pl.ds(rbase, RPT)])


EW5 = EP // 16
NWIN5 = EW5 // CW


CWV = 128
NWINV = EW5 // CWV


@_lazy_sc_kernel(
    out_type=jax.ShapeDtypeStruct((2, NP, 128), jnp.float32),
    scratch_types=[
        pltpu.VMEM((EW5,), jnp.int32),
        pltpu.VMEM((1, CWV), jnp.int32),
        pltpu.VMEM((CWV, 128), jnp.float32),
        pltpu.VMEM((CWV, 128), jnp.float32),
        pltpu.VMEM_SHARED((NP, 128), jnp.float32),
        pltpu.SemaphoreType.DMA,
        pltpu.SemaphoreType.DMA,
    ])
def _k5(src_h, dstp2_h, yb_h, g2_h, srcw, dstw, rows0, rows1, gacc,
        sem0, sem1):
    cid = lax.axis_index("c")
    sid = lax.axis_index("s")
    base = sid * EW5
    off = cid * NP
    pltpu.sync_copy(src_h.at[pl.ds(base, EW5)], srcw)
    offv = jnp.full((16,), 0, jnp.int32) + off

    def abody(j, _):
        srcw[pl.ds(j * 16, 16)] = srcw[pl.ds(j * 16, 16)] + offv
        return 0

    lax.fori_loop(0, EW5 // 16, abody, 0, unroll=4)
    rbase = sid * RPT
    pltpu.sync_copy(yb_h.at[pl.ds(off + rbase, RPT)],
                    gacc.at[pl.ds(rbase, RPT)])
    plsc.subcore_barrier()

    pltpu.async_copy(yb_h.at[srcw.at[pl.ds(0, CWV)]], rows0, sem0)
    pltpu.async_copy(yb_h.at[srcw.at[pl.ds(CWV, CWV)]], rows1, sem1)

    def gbody(g, _):
        for b, (rows, sem) in enumerate(((rows0, sem0), (rows1, sem1))):
            w = g * 2 + b
            pltpu.make_async_copy(yb_h.at[pl.ds(0, CWV)], rows, sem).wait()
            pltpu.sync_copy(dstp2_h.at[pl.ds(sid * NWINV + w, 1)], dstw)
            pltpu.sync_copy(rows, gacc.at[dstw.at[0]], add=True)

            @pl.when(w + 2 < NWINV)
            def _():
                pltpu.async_copy(
                    yb_h.at[srcw.at[pl.ds((w + 2) * CWV, CWV)]], rows, sem)
        return 0

    lax.fori_loop(0, NWINV // 2, gbody, 0)
    plsc.subcore_barrier()
    pltpu.sync_copy(gacc.at[pl.ds(rbase, RPT)],
                    g2_h.at[cid, pl.ds(rbase, RPT)])


def kernel(x, edge_index, W1, b1, W2, b2, g1, be1, g2, be2, g3, be3,
           fc1W, fc1b, fc2W, fc2b):
    f32 = jnp.float32
    kdrop = jax.random.key(42)
    ke, k1, k2 = jax.random.split(kdrop, 3)
    mask = jax.random.bernoulli(ke, 0.8, (E,))
    m1 = jax.random.bernoulli(k1, 0.1, (1, 1024)).astype(f32) * 10.0
    m2 = jax.random.bernoulli(k2, 0.1, (1, 1024)).astype(f32) * 10.0

    src = edge_index[0]
    dst = edge_index[1]
    npad = EP - E
    srcp = jnp.concatenate([src, jnp.arange(npad, dtype=jnp.int32)])
    dstpad = jnp.concatenate([dst, jnp.zeros((npad,), jnp.int32)])
    maskp = jnp.concatenate([mask.astype(jnp.int32),
                             jnp.zeros((npad,), jnp.int32)])

    xpad = jnp.pad(x, ((0, NP - N), (0, 28)))
    g1p = jnp.pad(g1, (0, 28)).reshape(1, 128)
    be1p = jnp.pad(be1, (0, 28)).reshape(1, 128)
    w1p = jnp.pad(W1, ((0, 28), (0, 0)))
    b1r = b1.reshape(1, 1024)
    g2r = g2.reshape(1, 1024)
    be2r = be2.reshape(1, 1024)
    b2r = b2.reshape(1, 1024)
    g3r = g3.reshape(1, 1024)
    be3r = be3.reshape(1, 1024)
    fc1br = fc1b.reshape(1, 1024)
    fc2wp = jnp.pad(fc2W, ((0, 0), (0, 28)))
    fc2bp = jnp.pad(fc2b, (0, 28)).reshape(1, 128)

    dstp, degp = _k1(dstpad, maskp)
    deg0 = degp[0].reshape(NP, 1)
    deg1 = degp[1].reshape(NP, 1)

    dinv, xp, s1, t1 = _t1_call(deg0, deg1, xpad, g1p, be1p)
    dstp2 = dstp.reshape(EP // CW, CW)

    zeros_np = jnp.zeros((NP, 128), f32)
    g1p_agg = _k2(srcp, dstp2, xp, zeros_np)
    g1a = g1p_agg[0]
    g1b = g1p_agg[1]

    z1, rs, s2, t2 = _t3_call(g1a, g1b, dinv, w1p, s1, t1, b1r, g2r, be2r)

    def strips(v):
        m = v.reshape(8, 1, 128)
        return [jnp.stack([m[j], m[j + 4]]) for j in range(4)]

    b2s = strips(b2)
    g3s = strips(g3)
    be3s = strips(be3)
    dstp2v = dstp.reshape(EP // CWV, CWV)
    pools = []
    for j in range(4):
        yb_j, c2_j = _t4_call(z1, s2, t2, dinv, W2, j)
        g2_j = _k5(srcp, dstp2v, yb_j.reshape(2 * NP, 128))
        pool_j, = _t6_call(g2_j, dinv, rs, c2_j, b2s[j], g3s[j], be3s[j], j)
        pools.append(pool_j)

    pool8 = jnp.concatenate(
        [jnp.concatenate([pools[j][s] for j in range(4)], axis=0)
         for s in range(2)], axis=0)
    poolr = pool8.reshape(1, 1024)

    out = _t7_call(poolr, m1, m2, fc1W, fc1br, fc2wp, fc2bp)
    return out[:, :100]


# speedup vs baseline: 11.0293x; 1.0184x over previous
import functools
import jax
import jax.numpy as jnp
from jax import lax
from jax.experimental import pallas as pl
from jax.experimental.pallas import tpu as pltpu
from jax.experimental.pallas import tpu_sc as plsc

N = 10000
NP = 10240
E = 160000
EPS = 1e-5
NINV = 1.0 / N

EP = 163840
NWK = 32
EW = EP // NWK
CWV = 128
RPT = NP // 16


def _t1_body(deg0, deg1, xpad, g1, be1, dinv_o, xp_o, s1_o, t1_o, asum, asq):
    i = pl.program_id(0)
    B = xpad.shape[0]
    deg = deg0[...] + deg1[...] + 1.0
    dinv = lax.rsqrt(deg)
    dinv_o[...] = dinv
    x = xpad[...]
    col = lax.broadcasted_iota(jnp.int32, (B, 128), 1)
    xp_o[...] = jnp.where(col == 127, dinv, dinv * x)

    @pl.when(i == 0)
    def _():
        asum[...] = jnp.zeros_like(asum)
        asq[...] = jnp.zeros_like(asq)

    asum[...] += jnp.sum(x, axis=0, keepdims=True)
    asq[...] += jnp.sum(x * x, axis=0, keepdims=True)

    @pl.when(i == pl.num_programs(0) - 1)
    def _():
        mu = asum[...] * NINV
        var = asq[...] * NINV - mu * mu
        s1 = g1[...] * lax.rsqrt(var + EPS)
        s1_o[...] = s1
        t1_o[...] = be1[...] - mu * s1


def _t1_call(deg0, deg1, xpad, g1, be1):
    B = 512
    grid = NP // B
    return pl.pallas_call(
        _t1_body,
        grid=(grid,),
        in_specs=[
            pl.BlockSpec((B, 1), lambda i: (i, 0)),
            pl.BlockSpec((B, 1), lambda i: (i, 0)),
            pl.BlockSpec((B, 128), lambda i: (i, 0)),
            pl.BlockSpec((1, 128), lambda i: (0, 0)),
            pl.BlockSpec((1, 128), lambda i: (0, 0)),
        ],
        out_specs=[
            pl.BlockSpec((B, 1), lambda i: (i, 0)),
            pl.BlockSpec((B, 128), lambda i: (i, 0)),
            pl.BlockSpec((1, 128), lambda i: (0, 0)),
            pl.BlockSpec((1, 128), lambda i: (0, 0)),
        ],
        out_shape=[
            jax.ShapeDtypeStruct((NP, 1), jnp.float32),
            jax.ShapeDtypeStruct((NP, 128), jnp.float32),
            jax.ShapeDtypeStruct((1, 128), jnp.float32),
            jax.ShapeDtypeStruct((1, 128), jnp.float32),
        ],
        scratch_shapes=[
            pltpu.VMEM((1, 128), jnp.float32),
            pltpu.VMEM((1, 128), jnp.float32),
        ],
    )(deg0, deg1, xpad, g1, be1)


def _t3_body(g1a, g1b, dinv, w1, s1, t1, b1, g2, be2,
             z1_o, rs_o, s2_o, t2_o, asum, asq):
    i = pl.program_id(0)
    B = g1a.shape[0]
    dv = dinv[...]
    g = g1a[...] + g1b[...]
    rs = g[:, 127:128]
    rs_o[...] = rs
    lhs = g * dv * s1[...]
    c1 = jnp.dot(t1[...], w1[...], preferred_element_type=jnp.float32)
    r = dv * rs
    z = jnp.dot(lhs, w1[...], preferred_element_type=jnp.float32)
    z = jnp.maximum(z + r * c1 + b1[...], 0.0)
    z1_o[...] = z

    @pl.when(i == 0)
    def _():
        asum[...] = jnp.zeros_like(asum)
        asq[...] = jnp.zeros_like(asq)

    row = i * B + lax.broadcasted_iota(jnp.int32, (B, 1), 0)
    zm = jnp.where(row < N, z, 0.0)
    asum[...] += jnp.sum(zm, axis=0, keepdims=True)
    asq[...] += jnp.sum(zm * zm, axis=0, keepdims=True)

    @pl.when(i == pl.num_programs(0) - 1)
    def _():
        mu = asum[...] * NINV
        var = asq[...] * NINV - mu * mu
        s2 = g2[...] * lax.rsqrt(var + EPS)
        s2_o[...] = s2
        t2_o[...] = be2[...] - mu * s2


def _t3_call(g1a, g1b, dinv, w1, s1, t1, b1, g2, be2):
    B = 512
    grid = NP // B
    return pl.pallas_call(
        _t3_body,
        grid=(grid,),
        in_specs=[
            pl.BlockSpec((B, 128), lambda i: (i, 0)),
            pl.BlockSpec((B, 128), lambda i: (i, 0)),
            pl.BlockSpec((B, 1), lambda i: (i, 0)),
            pl.BlockSpec((128, 1024), lambda i: (0, 0)),
            pl.BlockSpec((1, 128), lambda i: (0, 0)),
            pl.BlockSpec((1, 128), lambda i: (0, 0)),
            pl.BlockSpec((1, 1024), lambda i: (0, 0)),
            pl.BlockSpec((1, 1024), lambda i: (0, 0)),
            pl.BlockSpec((1, 1024), lambda i: (0, 0)),
        ],
        out_specs=[
            pl.BlockSpec((B, 1024), lambda i: (i, 0)),
            pl.BlockSpec((B, 1), lambda i: (i, 0)),
            pl.BlockSpec((1, 1024), lambda i: (0, 0)),
            pl.BlockSpec((1, 1024), lambda i: (0, 0)),
        ],
        out_shape=[
            jax.ShapeDtypeStruct((NP, 1024), jnp.float32),
            jax.ShapeDtypeStruct((NP, 1), jnp.float32),
            jax.ShapeDtypeStruct((1, 1024), jnp.float32),
            jax.ShapeDtypeStruct((1, 1024), jnp.float32),
        ],
        scratch_shapes=[
            pltpu.VMEM((1, 1024), jnp.float32),
            pltpu.VMEM((1, 1024), jnp.float32),
        ],
    )(g1a, g1b, dinv, w1, s1, t1, b1, g2, be2)


def _t4_body(z1, s2, t2, dinv, w2, yb_o, c2_o):
    zs = z1[...] * s2[...]
    w = w2[...]
    yb_o[0] = jnp.dot(zs, w, preferred_element_type=jnp.float32) * dinv[...]
    c2_o[0] = jnp.dot(t2[...], w, preferred_element_type=jnp.float32)


def _t4_call(z1, s2, t2, dinv, w2, j):
    B = 512
    gi = NP // B
    return pl.pallas_call(
        _t4_body,
        grid=(gi, 2),
        in_specs=[
            pl.BlockSpec((B, 1024), lambda i, s: (i, 0)),
            pl.BlockSpec((1, 1024), lambda i, s: (0, 0)),
            pl.BlockSpec((1, 1024), lambda i, s: (0, 0)),
            pl.BlockSpec((B, 1), lambda i, s: (i, 0)),
            pl.BlockSpec((1024, 128), lambda i, s: (0, j + 4 * s)),
        ],
        out_specs=[
            pl.BlockSpec((1, B, 128), lambda i, s: (s, i, 0)),
            pl.BlockSpec((1, 1, 128), lambda i, s: (s, 0, 0)),
        ],
        out_shape=[
            jax.ShapeDtypeStruct((2, NP, 128), jnp.float32),
            jax.ShapeDtypeStruct((2, 1, 128), jnp.float32),
        ],
    )(z1, s2, t2, dinv, w2)


def _t6_body(g2, dinv, rs, c2, b2, g3, be3, pool_o, ssum, ssq, smax):
    i = pl.program_id(0)
    B = dinv.shape[0]
    dv = dinv[...]
    r = dv * rs[...]
    row = i * B + lax.broadcasted_iota(jnp.int32, (B, 1), 0)
    valid = row < N

    @pl.when(i == 0)
    def _():
        ssum[...] = jnp.zeros_like(ssum)
        ssq[...] = jnp.zeros_like(ssq)
        smax[...] = jnp.full_like(smax, -3e38)

    for s in range(2):
        z2 = jnp.maximum(dv * g2[s] + r * c2[s] + b2[s], 0.0)
        zm = jnp.where(valid, z2, 0.0)
        ssum[pl.ds(s, 1), :] += jnp.sum(zm, axis=0, keepdims=True)
        ssq[pl.ds(s, 1), :] += jnp.sum(zm * zm, axis=0, keepdims=True)
        zx = jnp.where(valid, z2, -3e38)
        smax[pl.ds(s, 1), :] = jnp.maximum(
            smax[pl.ds(s, 1), :], jnp.max(zx, axis=0, keepdims=True))

    @pl.when(i == pl.num_programs(0) - 1)
    def _():
        for s in range(2):
            mu = ssum[pl.ds(s, 1), :] * NINV
            var = ssq[pl.ds(s, 1), :] * NINV - mu * mu
            pool_o[s] = ((smax[pl.ds(s, 1), :] - mu) * lax.rsqrt(var + EPS)
                         * g3[s] + be3[s])


def _t6_call(g2, dinv, rs, c2, b2, g3, be3, j):
    B = 512
    grid = NP // B
    return pl.pallas_call(
        _t6_body,
        grid=(grid,),
        in_specs=[
            pl.BlockSpec((2, B, 128), lambda i: (0, i, 0)),
            pl.BlockSpec((B, 1), lambda i: (i, 0)),
            pl.BlockSpec((B, 1), lambda i: (i, 0)),
            pl.BlockSpec((2, 1, 128), lambda i: (0, 0, 0)),
            pl.BlockSpec((2, 1, 128), lambda i: (0, 0, 0)),
            pl.BlockSpec((2, 1, 128), lambda i: (0, 0, 0)),
            pl.BlockSpec((2, 1, 128), lambda i: (0, 0, 0)),
        ],
        out_specs=[pl.BlockSpec((2, 1, 128), lambda i: (0, 0, 0))],
        out_shape=[jax.ShapeDtypeStruct((2, 1, 128), jnp.float32)],
        scratch_shapes=[
            pltpu.VMEM((2, 128), jnp.float32),
            pltpu.VMEM((2, 128), jnp.float32),
            pltpu.VMEM((2, 128), jnp.float32),
        ],
    )(g2, dinv, rs, c2, b2, g3, be3)


def _t7_body(pool, m1, m2, fc1w, fc1b, fc2w, fc2b, out_o):
    h = jnp.dot(pool[...] * m1[...], fc1w[...], preferred_element_type=jnp.float32)
    h = jnp.maximum(h + fc1b[...], 0.0)
    o = jnp.dot(h * m2[...], fc2w[...], preferred_element_type=jnp.float32)
    o = o + fc2b[...]
    out_o[...] = 1.0 / (1.0 + jnp.exp(-o))


def _t7_call(pool, m1, m2, fc1w, fc1b, fc2w, fc2b):
    return pl.pallas_call(
        _t7_body,
        out_shape=jax.ShapeDtypeStruct((1, 128), jnp.float32),
    )(pool, m1, m2, fc1w, fc1b, fc2w, fc2b)


@functools.cache
def _sc_mesh():
    return plsc.VectorSubcoreMesh(core_axis_name="c", subcore_axis_name="s",
                                  num_cores=2, num_subcores=16)


def _lazy_sc_kernel(out_type, scratch_types):
    def deco(body):
        @functools.cache
        def build():
            return pl.kernel(
                body, out_type=out_type, mesh=_sc_mesh(),
                scratch_types=scratch_types,
                compiler_params=pltpu.CompilerParams(
                    needs_layout_passes=False,
                    use_tc_tiling_on_sc=False))

        def call(*args):
            return build()(*args)
        return call
    return deco


def _zero_ref(ref, n):
    z = jnp.zeros((16,), jnp.float32)

    def body(i, _):
        ref[pl.ds(i * 16, 16)] = z
        return 0

    lax.fori_loop(0, n // 16, body, 0, unroll=4)


def _tile_reduce(part_v, spm, out_h, cid, sid, tmp, acc2, sem):
    pltpu.sync_copy(part_v, spm.at[sid])
    plsc.subcore_barrier()
    _zero_ref(acc2, RPT)
    base = sid * RPT
    for p in range(16):
        pltpu.sync_copy(spm.at[p, pl.ds(base, RPT)], tmp)

        def body(j, _):
            acc2[pl.ds(j * 16, 16)] += tmp[pl.ds(j * 16, 16)]
            return 0

        lax.fori_loop(0, RPT // 16, body, 0, unroll=4)
    pltpu.sync_copy(acc2, out_h.at[cid, pl.ds(base, RPT)])


@_lazy_sc_kernel(
    out_type=[jax.ShapeDtypeStruct((EP,), jnp.int32),
              jax.ShapeDtypeStruct((2, NP), jnp.float32)],
    scratch_types=[
        pltpu.VMEM((EW,), jnp.int32),
        pltpu.VMEM((EW,), jnp.int32),
        pltpu.VMEM((EW,), jnp.int32),
        pltpu.VMEM((NP,), jnp.float32),
        pltpu.VMEM_SHARED((16, NP), jnp.float32),
        pltpu.VMEM((RPT,), jnp.float32),
        pltpu.VMEM((RPT,), jnp.float32),
        pltpu.SemaphoreType.DMA,
    ])
def _k1(dst_h, mask_h, dstp_h, degp_h, dstw, maskw, dpw, acc, spm, tmp, acc2, sem):
    cid = lax.axis_index("c")
    sid = lax.axis_index("s")
    wid = sid * 2 + cid
    base = wid * EW
    _zero_ref(acc, NP)
    pltpu.sync_copy(dst_h.at[pl.ds(base, EW)], dstw)
    pltpu.sync_copy(mask_h.at[pl.ds(base, EW)], maskw)
    dummy = N + ((lax.iota(jnp.int32, 16) + wid * 16) & 63)

    def body(j, _):
        dv = dstw[pl.ds(j * 16, 16)]
        mv = maskw[pl.ds(j * 16, 16)]
        dp = jnp.where(mv > 0, dv, dummy)
        dpw[pl.ds(j * 16, 16)] = dp
        cnt, last = plsc.scan_count(dp)
        plsc.addupdate_scatter(acc, [dp], cnt.astype(jnp.float32), mask=last)
        return 0

    lax.fori_loop(0, EW // 16, body, 0, unroll=2)
    pltpu.sync_copy(dpw, dstp_h.at[pl.ds(base, EW)])
    _tile_reduce(acc, spm, degp_h, cid, sid, tmp, acc2, sem)


NWIN2 = EW // CWV


@_lazy_sc_kernel(
    out_type=jax.ShapeDtypeStruct((2, NP, 128), jnp.float32),
    scratch_types=[
        pltpu.VMEM((EW,), jnp.int32),
        pltpu.VMEM((1, CWV), jnp.int32),
        pltpu.VMEM((CWV, 128), jnp.float32),
        pltpu.VMEM((CWV, 128), jnp.float32),
        pltpu.VMEM_SHARED((NP, 128), jnp.float32),
        pltpu.SemaphoreType.DMA,
        pltpu.SemaphoreType.DMA,
    ])
def _k2(src_h, dstp2_h, xp_h, zeros_h, g1p_h, srcw, dstw, rows0, rows1, gacc,
        sem0, sem1):
    cid = lax.axis_index("c")
    sid = lax.axis_index("s")
    wid = sid * 2 + cid
    base = wid * EW
    pltpu.sync_copy(src_h.at[pl.ds(base, EW)], srcw)
    rbase = sid * RPT

    @pl.when(cid == 0)
    def _():
        pltpu.sync_copy(xp_h.at[pl.ds(rbase, RPT)], gacc.at[pl.ds(rbase, RPT)])

    @pl.when(cid == 1)
    def _():
        pltpu.sync_copy(zeros_h.at[pl.ds(rbase, RPT)], gacc.at[pl.ds(rbase, RPT)])

    plsc.subcore_barrier()

    pltpu.async_copy(xp_h.at[srcw.at[pl.ds(0, CWV)]], rows0, sem0)
    pltpu.async_copy(xp_h.at[srcw.at[pl.ds(CWV, CWV)]], rows1, sem1)

    def gbody(g, _):
        for b, (rows, sem) in enumerate(((rows0, sem0), (rows1, sem1))):
            w = g * 2 + b
            pltpu.make_async_copy(xp_h.at[pl.ds(0, CWV)], rows, sem).wait()
            pltpu.sync_copy(dstp2_h.at[pl.ds(wid * NWIN2 + w, 1)], dstw)
            pltpu.sync_copy(rows, gacc.at[dstw.at[0]], add=True)

            @pl.when(w + 2 < NWIN2)
            def _():
                pltpu.async_copy(
                    xp_h.at[srcw.at[pl.ds((w + 2) * CWV, CWV)]], rows, sem)
        return 0

    lax.fori_loop(0, NWIN2 // 2, gbody, 0)
    plsc.subcore_barrier()
    pltpu.sync_copy(gacc.at[pl.ds(rbase, RPT)],
                    g1p_h.at[cid, pl.ds(rbase, RPT)])


EW5 = EP // 16


NWINV = EW5 // CWV


@_lazy_sc_kernel(
    out_type=jax.ShapeDtypeStruct((2, NP, 128), jnp.float32),
    scratch_types=[
        pltpu.VMEM((EW5,), jnp.int32),
        pltpu.VMEM((1, CWV), jnp.int32),
        pltpu.VMEM((CWV, 128), jnp.float32),
        pltpu.VMEM((CWV, 128), jnp.float32),
        pltpu.VMEM_SHARED((NP, 128), jnp.float32),
        pltpu.SemaphoreType.DMA,
        pltpu.SemaphoreType.DMA,
    ])
def _k5(src_h, dstp2_h, yb_h, g2_h, srcw, dstw, rows0, rows1, gacc,
        sem0, sem1):
    cid = lax.axis_index("c")
    sid = lax.axis_index("s")
    base = sid * EW5
    off = cid * NP
    pltpu.sync_copy(src_h.at[pl.ds(base, EW5)], srcw)
    offv = jnp.full((16,), 0, jnp.int32) + off

    def abody(j, _):
        srcw[pl.ds(j * 16, 16)] = srcw[pl.ds(j * 16, 16)] + offv
        return 0

    lax.fori_loop(0, EW5 // 16, abody, 0, unroll=4)
    rbase = sid * RPT
    pltpu.sync_copy(yb_h.at[pl.ds(off + rbase, RPT)],
                    gacc.at[pl.ds(rbase, RPT)])
    plsc.subcore_barrier()

    pltpu.async_copy(yb_h.at[srcw.at[pl.ds(0, CWV)]], rows0, sem0)
    pltpu.async_copy(yb_h.at[srcw.at[pl.ds(CWV, CWV)]], rows1, sem1)

    def gbody(g, _):
        for b, (rows, sem) in enumerate(((rows0, sem0), (rows1, sem1))):
            w = g * 2 + b
            pltpu.make_async_copy(yb_h.at[pl.ds(0, CWV)], rows, sem).wait()
            pltpu.sync_copy(dstp2_h.at[pl.ds(sid * NWINV + w, 1)], dstw)
            pltpu.sync_copy(rows, gacc.at[dstw.at[0]], add=True)

            @pl.when(w + 2 < NWINV)
            def _():
                pltpu.async_copy(
                    yb_h.at[srcw.at[pl.ds((w + 2) * CWV, CWV)]], rows, sem)
        return 0

    lax.fori_loop(0, NWINV // 2, gbody, 0)
    plsc.subcore_barrier()
    pltpu.sync_copy(gacc.at[pl.ds(rbase, RPT)],
                    g2_h.at[cid, pl.ds(rbase, RPT)])


def kernel(x, edge_index, W1, b1, W2, b2, g1, be1, g2, be2, g3, be3,
           fc1W, fc1b, fc2W, fc2b):
    f32 = jnp.float32
    kdrop = jax.random.key(42)
    ke, k1, k2 = jax.random.split(kdrop, 3)
    mask = jax.random.bernoulli(ke, 0.8, (E,))
    m1 = jax.random.bernoulli(k1, 0.1, (1, 1024)).astype(f32) * 10.0
    m2 = jax.random.bernoulli(k2, 0.1, (1, 1024)).astype(f32) * 10.0

    src = edge_index[0]
    dst = edge_index[1]
    npad = EP - E
    srcp = jnp.concatenate([src, jnp.arange(npad, dtype=jnp.int32)])
    dstpad = jnp.concatenate([dst, jnp.zeros((npad,), jnp.int32)])
    maskp = jnp.concatenate([mask.astype(jnp.int32),
                             jnp.zeros((npad,), jnp.int32)])

    xpad = jnp.pad(x, ((0, NP - N), (0, 28)))
    g1p = jnp.pad(g1, (0, 28)).reshape(1, 128)
    be1p = jnp.pad(be1, (0, 28)).reshape(1, 128)
    w1p = jnp.pad(W1, ((0, 28), (0, 0)))
    b1r = b1.reshape(1, 1024)
    g2r = g2.reshape(1, 1024)
    be2r = be2.reshape(1, 1024)
    b2r = b2.reshape(1, 1024)
    g3r = g3.reshape(1, 1024)
    be3r = be3.reshape(1, 1024)
    fc1br = fc1b.reshape(1, 1024)
    fc2wp = jnp.pad(fc2W, ((0, 0), (0, 28)))
    fc2bp = jnp.pad(fc2b, (0, 28)).reshape(1, 128)

    dstp, degp = _k1(dstpad, maskp)
    deg0 = degp[0].reshape(NP, 1)
    deg1 = degp[1].reshape(NP, 1)

    dinv, xp, s1, t1 = _t1_call(deg0, deg1, xpad, g1p, be1p)
    dstp2v = dstp.reshape(EP // CWV, CWV)

    zeros_np = jnp.zeros((NP, 128), f32)
    g1p_agg = _k2(srcp, dstp2v, xp, zeros_np)
    g1a = g1p_agg[0]
    g1b = g1p_agg[1]

    z1, rs, s2, t2 = _t3_call(g1a, g1b, dinv, w1p, s1, t1, b1r, g2r, be2r)

    def strips(v):
        m = v.reshape(8, 1, 128)
        return [jnp.stack([m[j], m[j + 4]]) for j in range(4)]

    b2s = strips(b2)
    g3s = strips(g3)
    be3s = strips(be3)
    pools = []
    for j in range(4):
        yb_j, c2_j = _t4_call(z1, s2, t2, dinv, W2, j)
        g2_j = _k5(srcp, dstp2v, yb_j.reshape(2 * NP, 128))
        pool_j, = _t6_call(g2_j, dinv, rs, c2_j, b2s[j], g3s[j], be3s[j], j)
        pools.append(pool_j)

    pool8 = jnp.concatenate(
        [jnp.concatenate([pools[j][s] for j in range(4)], axis=0)
         for s in range(2)], axis=0)
    poolr = pool8.reshape(1, 1024)

    out = _t7_call(poolr, m1, m2, fc1W, fc1br, fc2wp, fc2bp)
    return out[:, :100]
